# Initial kernel scaffold; baseline (speedup 1.0000x reference)
import jax
import jax.numpy as jnp
from jax.experimental import pallas as pl


def kernel(x, edge_index, edge_weight, W, b):
    raise NotImplementedError("write your pallas kernel here")


# speedup vs baseline: 203.6217x; 203.6217x over previous
import dataclasses
import functools

import jax
import jax.numpy as jnp
from jax import lax
from jax.experimental import pallas as pl
from jax.experimental.pallas import tpu as pltpu
from jax.experimental.pallas import tpu_sc as plsc

NC = 2
NS = 16
LANES = 16

_HIST_WIN = 8000
_AGG_WIN = 2000


def _sc_compiler_params():
    cp = pltpu.CompilerParams()
    if "needs_layout_passes" in pltpu.CompilerParams.__dataclass_fields__:
        cp = dataclasses.replace(cp, needs_layout_passes=False)
    return cp


def _sc_degree_hist(eflat, e, n_pad, interpret=False):
    ept = e // NS
    win = _HIST_WIN
    assert e % NS == 0 and ept % win == 0 and win % 80 == 0
    nwin = ept // win
    assert nwin % 2 == 0
    mesh = plsc.VectorSubcoreMesh(core_axis_name="c", subcore_axis_name="s",
                                  num_cores=NC, num_subcores=NS)

    @functools.partial(
        pl.kernel,
        out_type=jax.ShapeDtypeStruct((NC * NS * n_pad,), jnp.float32),
        mesh=mesh,
        interpret=interpret,
        compiler_params=_sc_compiler_params(),
        scratch_types=[
            pltpu.VMEM((n_pad,), jnp.float32),
            pltpu.VMEM((win,), jnp.int32),
            pltpu.VMEM((win,), jnp.int32),
            pltpu.SemaphoreType.DMA,
            pltpu.SemaphoreType.DMA,
        ],
    )
    def hist_kernel(eidx_hbm, out_hbm, hist, ibuf0, ibuf1, sem0, sem1):
        c = lax.axis_index("c")
        s = lax.axis_index("s")
        ibufs = (ibuf0, ibuf1)
        sems = (sem0, sem1)
        zeros = jnp.zeros((LANES,), jnp.float32)
        ones = jnp.full((LANES,), 1.0, jnp.float32)

        @pl.loop(0, n_pad, step=8 * LANES)
        def _(i):
            for j in range(8):
                hist[pl.ds(i + j * LANES, LANES)] = zeros

        base = c * e + s * ept
        for k in range(2):
            pltpu.async_copy(
                eidx_hbm.at[pl.ds(base + k * win, win)], ibufs[k], sems[k]
            )

        @pl.loop(0, nwin, step=2)
        def _(w):
            for k in range(2):
                pltpu.make_async_copy(
                    eidx_hbm.at[pl.ds(base, win)], ibufs[k], sems[k]
                ).wait()

                @pl.loop(0, win, step=5 * LANES)
                def _(i):
                    for j in range(5):
                        idx = ibufs[k][pl.ds(i + j * LANES, LANES)]
                        plsc.addupdate_scatter(hist, [idx], ones)

                nxt = w + k + 2

                @pl.when(nxt < nwin)
                def _():
                    pltpu.async_copy(
                        eidx_hbm.at[pl.ds(base + nxt * win, win)],
                        ibufs[k],
                        sems[k],
                    )

        pltpu.sync_copy(hist, out_hbm.at[pl.ds((c * NS + s) * n_pad, n_pad)])

    return hist_kernel(eflat)


def _sc_aggregate(feat_pad, eflat, edge_weight, e, n_pad, interpret=False):
    epw = e // (NC * NS)
    win = _AGG_WIN
    assert e % (NC * NS) == 0 and epw % win == 0 and win % 80 == 0
    nwin = epw // win
    assert nwin % 2 == 0 and n_pad % (8 * NS) == 0
    slab = n_pad // NS
    mesh = plsc.VectorSubcoreMesh(core_axis_name="c", subcore_axis_name="s",
                                  num_cores=NC, num_subcores=NS)

    @functools.partial(
        pl.kernel,
        out_type=jax.ShapeDtypeStruct((NC * NS * n_pad,), jnp.float32),
        mesh=mesh,
        interpret=interpret,
        compiler_params=_sc_compiler_params(),
        scratch_types=[
            pltpu.VMEM((n_pad,), jnp.float32),
            pltpu.VMEM_SHARED((n_pad,), jnp.float32),
            [pltpu.VMEM((win,), jnp.int32)] * 2,
            [pltpu.VMEM((win,), jnp.int32)] * 2,
            [pltpu.VMEM((win,), jnp.float32)] * 2,
            [pltpu.VMEM((win,), jnp.float32)] * 2,
            [pltpu.SemaphoreType.DMA] * 2,
            [pltpu.SemaphoreType.DMA] * 2,
            [pltpu.SemaphoreType.DMA] * 2,
            [pltpu.SemaphoreType.DMA] * 2,
        ],
    )
    def agg_kernel(feat_hbm, eidx_hbm, ew_hbm, out_hbm, h, featsh, sbuf, dbuf,
                   wbuf, fbuf, ssems, dsems, wsems, gsems):
        c = lax.axis_index("c")
        s = lax.axis_index("s")
        zeros = jnp.zeros((LANES,), jnp.float32)

        pltpu.async_copy(
            feat_hbm.at[pl.ds(s * slab, slab)],
            featsh.at[pl.ds(s * slab, slab)],
            gsems[0],
        )

        @pl.loop(0, n_pad, step=8 * LANES)
        def _(i):
            for j in range(8):
                h[pl.ds(i + j * LANES, LANES)] = zeros

        pltpu.make_async_copy(
            feat_hbm.at[pl.ds(0, slab)], featsh.at[pl.ds(0, slab)], gsems[0]
        ).wait()
        plsc.subcore_barrier()

        base = (c * NS + s) * epw

        def start_window(w, k):
            pltpu.async_copy(
                eidx_hbm.at[pl.ds(base + w * win, win)], sbuf[k], ssems[k]
            )
            pltpu.async_copy(
                eidx_hbm.at[pl.ds(e + base + w * win, win)], dbuf[k],
                dsems[k]
            )
            pltpu.async_copy(
                ew_hbm.at[pl.ds(base + w * win, win)], wbuf[k], wsems[k]
            )

        def wait_src(k):
            pltpu.make_async_copy(
                eidx_hbm.at[pl.ds(base, win)], sbuf[k], ssems[k]
            ).wait()

        def start_gather(k):
            pltpu.async_copy(featsh.at[sbuf[k]], fbuf[k], gsems[k])

        def wait_gather(k):
            pltpu.make_async_copy(
                featsh.at[sbuf[k]], fbuf[k], gsems[k]
            ).wait()

        start_window(0, 0)
        start_window(1, 1)
        wait_src(0)
        start_gather(0)

        @pl.loop(0, nwin, step=2)
        def _(w):
            for k in range(2):
                cur = w + k
                nxt = cur + 1

                @pl.when(nxt < nwin)
                def _():
                    wait_src(1 - k)
                    start_gather(1 - k)

                wait_gather(k)
                pltpu.make_async_copy(
                    eidx_hbm.at[pl.ds(base, win)], dbuf[k], dsems[k]
                ).wait()
                pltpu.make_async_copy(
                    ew_hbm.at[pl.ds(base, win)], wbuf[k], wsems[k]
                ).wait()

                @pl.loop(0, win, step=5 * LANES)
                def _(i):
                    for j in range(5):
                        sl = pl.ds(i + j * LANES, LANES)
                        msg = fbuf[k][sl] * wbuf[k][sl]
                        plsc.addupdate_scatter(h, [dbuf[k][sl]], msg)

                @pl.when(cur + 2 < nwin)
                def _():
                    start_window(cur + 2, k)

        pltpu.sync_copy(h, out_hbm.at[pl.ds((c * NS + s) * n_pad, n_pad)])

    return agg_kernel(feat_pad, eflat, edge_weight)


def _rsqrt_exactish(x):
    y = jax.lax.rsqrt(x)
    y = y * (1.5 - 0.5 * x * y * y)
    y = y * (1.5 - 0.5 * x * y * y)
    return y


def _tc_feat(dp2, x0r, x1r, w_flat, n_pad, interpret=False):
    rows = n_pad // 128

    def body(w_ref, dp_ref, x0_ref, x1_ref, feat_ref, ir_ref):
        od = dp_ref[0:rows, :]
        for p in range(1, NS):
            od = od + dp_ref[p * rows:(p + 1) * rows, :]
        idg = dp_ref[NS * rows:(NS + 1) * rows, :]
        for p in range(NS + 1, 2 * NS):
            idg = idg + dp_ref[p * rows:(p + 1) * rows, :]
        od = jnp.maximum(od, 1.0)
        idg = jnp.maximum(idg, 1.0)
        xw = x0_ref[...] * w_ref[0, 0] + x1_ref[...] * w_ref[0, 1]
        feat_ref[...] = xw * _rsqrt_exactish(od)
        ir_ref[...] = _rsqrt_exactish(idg)

    return pl.pallas_call(
        body,
        out_shape=(
            jax.ShapeDtypeStruct((rows, 128), jnp.float32),
            jax.ShapeDtypeStruct((rows, 128), jnp.float32),
        ),
        in_specs=[
            pl.BlockSpec(memory_space=pltpu.SMEM),
            pl.BlockSpec(memory_space=pltpu.VMEM),
            pl.BlockSpec(memory_space=pltpu.VMEM),
            pl.BlockSpec(memory_space=pltpu.VMEM),
        ],
        interpret=interpret,
    )(w_flat, dp2, x0r, x1r)


def _tc_final(hp2, ir2, b, n_pad, interpret=False):
    rows = n_pad // 128

    def body(b_ref, hp_ref, ir_ref, o_ref):
        hsum = hp_ref[0:rows, :]
        for p in range(1, NC * NS):
            hsum = hsum + hp_ref[p * rows:(p + 1) * rows, :]
        o_ref[...] = hsum * ir_ref[...] + b_ref[0, 0]

    return pl.pallas_call(
        body,
        out_shape=jax.ShapeDtypeStruct((rows, 128), jnp.float32),
        in_specs=[
            pl.BlockSpec(memory_space=pltpu.SMEM),
            pl.BlockSpec(memory_space=pltpu.VMEM),
            pl.BlockSpec(memory_space=pltpu.VMEM),
        ],
        interpret=interpret,
    )(b.reshape(1, 1), hp2, ir2)


def kernel(x, edge_index, edge_weight, W, b):
    n = x.shape[0]
    e = edge_index.shape[1]
    n_pad = ((n + 1023) // 1024) * 1024
    rows = n_pad // 128
    pad = n_pad - n
    x0r = jnp.pad(x[:, 0], (0, pad)).reshape(rows, 128)
    x1r = jnp.pad(x[:, 1], (0, pad)).reshape(rows, 128)
    w_flat = W.reshape(1, 2)
    eflat = edge_index.reshape(2 * e)

    dp = _sc_degree_hist(eflat, e, n_pad)
    feat2, ir2 = _tc_feat(dp.reshape(NC * NS * rows, 128), x0r, x1r, w_flat,
                          n_pad)
    hp = _sc_aggregate(feat2.reshape(n_pad), eflat, edge_weight, e, n_pad)
    out2 = _tc_final(hp.reshape(NC * NS * rows, 128), ir2, b, n_pad)
    return out2.reshape(1, n_pad)[:, :n]


# speedup vs baseline: 319.3069x; 1.5681x over previous
import dataclasses
import functools

import jax
import jax.numpy as jnp
from jax import lax
from jax.experimental import pallas as pl
from jax.experimental.pallas import tpu as pltpu
from jax.experimental.pallas import tpu_sc as plsc

NC = 2
NS = 16
LANES = 16

_HIST_WIN = 8000
_AGG_WIN = 2000


def _sc_compiler_params():
    cp = pltpu.CompilerParams()
    if "needs_layout_passes" in pltpu.CompilerParams.__dataclass_fields__:
        cp = dataclasses.replace(cp, needs_layout_passes=False)
    return cp


def _sc_degree_hist(eflat, e, n_pad, interpret=False):
    ept = e // NS
    win = _HIST_WIN
    assert e % NS == 0 and ept % win == 0 and win % 80 == 0
    nwin = ept // win
    assert nwin % 2 == 0
    mesh = plsc.VectorSubcoreMesh(core_axis_name="c", subcore_axis_name="s",
                                  num_cores=NC, num_subcores=NS)

    @functools.partial(
        pl.kernel,
        out_type=jax.ShapeDtypeStruct((NC * NS * n_pad,), jnp.float32),
        mesh=mesh,
        interpret=interpret,
        compiler_params=_sc_compiler_params(),
        scratch_types=[
            pltpu.VMEM((n_pad,), jnp.float32),
            pltpu.VMEM((win,), jnp.int32),
            pltpu.VMEM((win,), jnp.int32),
            pltpu.SemaphoreType.DMA,
            pltpu.SemaphoreType.DMA,
        ],
    )
    def hist_kernel(eidx_hbm, out_hbm, hist, ibuf0, ibuf1, sem0, sem1):
        c = lax.axis_index("c")
        s = lax.axis_index("s")
        ibufs = (ibuf0, ibuf1)
        sems = (sem0, sem1)
        zeros = jnp.zeros((LANES,), jnp.float32)
        ones = jnp.full((LANES,), 1.0, jnp.float32)

        @plsc.parallel_loop(0, n_pad, step=LANES, unroll=8)
        def _(i):
            hist[pl.ds(i, LANES)] = zeros

        base = c * e + s * ept
        for k in range(2):
            pltpu.async_copy(
                eidx_hbm.at[pl.ds(base + k * win, win)], ibufs[k], sems[k]
            )

        @pl.loop(0, nwin, step=2)
        def _(w):
            for k in range(2):
                pltpu.make_async_copy(
                    eidx_hbm.at[pl.ds(base, win)], ibufs[k], sems[k]
                ).wait()

                @plsc.parallel_loop(0, win, step=LANES, unroll=10)
                def _(i):
                    idx = ibufs[k][pl.ds(i, LANES)]
                    plsc.addupdate_scatter(hist, [idx], ones)

                nxt = w + k + 2

                @pl.when(nxt < nwin)
                def _():
                    pltpu.async_copy(
                        eidx_hbm.at[pl.ds(base + nxt * win, win)],
                        ibufs[k],
                        sems[k],
                    )

        pltpu.sync_copy(hist, out_hbm.at[pl.ds((c * NS + s) * n_pad, n_pad)])

    return hist_kernel(eflat)


def _sc_aggregate(feat_pad, eflat, edge_weight, e, n_pad, interpret=False):
    epw = e // (NC * NS)
    win = _AGG_WIN
    assert e % (NC * NS) == 0 and epw % win == 0 and win % 80 == 0
    nwin = epw // win
    assert nwin % 2 == 0 and n_pad % (8 * NS) == 0
    slab = n_pad // NS
    mesh = plsc.VectorSubcoreMesh(core_axis_name="c", subcore_axis_name="s",
                                  num_cores=NC, num_subcores=NS)

    @functools.partial(
        pl.kernel,
        out_type=jax.ShapeDtypeStruct((NC * NS * n_pad,), jnp.float32),
        mesh=mesh,
        interpret=interpret,
        compiler_params=_sc_compiler_params(),
        scratch_types=[
            pltpu.VMEM((n_pad,), jnp.float32),
            pltpu.VMEM_SHARED((n_pad,), jnp.float32),
            [pltpu.VMEM((win,), jnp.int32)] * 2,
            [pltpu.VMEM((win,), jnp.int32)] * 2,
            [pltpu.VMEM((win,), jnp.float32)] * 2,
            [pltpu.VMEM((win,), jnp.float32)] * 2,
            [pltpu.SemaphoreType.DMA] * 2,
            [pltpu.SemaphoreType.DMA] * 2,
            [pltpu.SemaphoreType.DMA] * 2,
            [pltpu.SemaphoreType.DMA] * 2,
        ],
    )
    def agg_kernel(feat_hbm, eidx_hbm, ew_hbm, out_hbm, h, featsh, sbuf, dbuf,
                   wbuf, fbuf, ssems, dsems, wsems, gsems):
        c = lax.axis_index("c")
        s = lax.axis_index("s")
        zeros = jnp.zeros((LANES,), jnp.float32)

        pltpu.async_copy(
            feat_hbm.at[pl.ds(s * slab, slab)],
            featsh.at[pl.ds(s * slab, slab)],
            gsems[0],
        )

        @plsc.parallel_loop(0, n_pad, step=LANES, unroll=8)
        def _(i):
            h[pl.ds(i, LANES)] = zeros

        pltpu.make_async_copy(
            feat_hbm.at[pl.ds(0, slab)], featsh.at[pl.ds(0, slab)], gsems[0]
        ).wait()
        plsc.subcore_barrier()

        base = (c * NS + s) * epw

        def start_window(w, k):
            pltpu.async_copy(
                eidx_hbm.at[pl.ds(base + w * win, win)], sbuf[k], ssems[k]
            )
            pltpu.async_copy(
                eidx_hbm.at[pl.ds(e + base + w * win, win)], dbuf[k],
                dsems[k]
            )
            pltpu.async_copy(
                ew_hbm.at[pl.ds(base + w * win, win)], wbuf[k], wsems[k]
            )

        def wait_src(k):
            pltpu.make_async_copy(
                eidx_hbm.at[pl.ds(base, win)], sbuf[k], ssems[k]
            ).wait()

        def start_gather(k):
            pltpu.async_copy(featsh.at[sbuf[k]], fbuf[k], gsems[k])

        def wait_gather(k):
            pltpu.make_async_copy(
                featsh.at[sbuf[k]], fbuf[k], gsems[k]
            ).wait()

        start_window(0, 0)
        start_window(1, 1)
        wait_src(0)
        start_gather(0)

        @pl.loop(0, nwin, step=2)
        def _(w):
            for k in range(2):
                cur = w + k
                nxt = cur + 1

                @pl.when(nxt < nwin)
                def _():
                    wait_src(1 - k)
                    start_gather(1 - k)

                wait_gather(k)
                pltpu.make_async_copy(
                    eidx_hbm.at[pl.ds(base, win)], dbuf[k], dsems[k]
                ).wait()
                pltpu.make_async_copy(
                    ew_hbm.at[pl.ds(base, win)], wbuf[k], wsems[k]
                ).wait()

                @plsc.parallel_loop(0, win, step=LANES, unroll=5)
                def _(i):
                    sl = pl.ds(i, LANES)
                    msg = fbuf[k][sl] * wbuf[k][sl]
                    plsc.addupdate_scatter(h, [dbuf[k][sl]], msg)

                @pl.when(cur + 2 < nwin)
                def _():
                    start_window(cur + 2, k)

        pltpu.sync_copy(h, out_hbm.at[pl.ds((c * NS + s) * n_pad, n_pad)])

    return agg_kernel(feat_pad, eflat, edge_weight)


def _rsqrt_exactish(x):
    y = jax.lax.rsqrt(x)
    y = y * (1.5 - 0.5 * x * y * y)
    y = y * (1.5 - 0.5 * x * y * y)
    return y


def _tc_feat(dp2, x0r, x1r, w_flat, n_pad, interpret=False):
    rows = n_pad // 128

    def body(w_ref, dp_ref, x0_ref, x1_ref, feat_ref, ir_ref):
        od = dp_ref[0:rows, :]
        for p in range(1, NS):
            od = od + dp_ref[p * rows:(p + 1) * rows, :]
        idg = dp_ref[NS * rows:(NS + 1) * rows, :]
        for p in range(NS + 1, 2 * NS):
            idg = idg + dp_ref[p * rows:(p + 1) * rows, :]
        od = jnp.maximum(od, 1.0)
        idg = jnp.maximum(idg, 1.0)
        xw = x0_ref[...] * w_ref[0, 0] + x1_ref[...] * w_ref[0, 1]
        feat_ref[...] = xw * _rsqrt_exactish(od)
        ir_ref[...] = _rsqrt_exactish(idg)

    return pl.pallas_call(
        body,
        out_shape=(
            jax.ShapeDtypeStruct((rows, 128), jnp.float32),
            jax.ShapeDtypeStruct((rows, 128), jnp.float32),
        ),
        in_specs=[
            pl.BlockSpec(memory_space=pltpu.SMEM),
            pl.BlockSpec(memory_space=pltpu.VMEM),
            pl.BlockSpec(memory_space=pltpu.VMEM),
            pl.BlockSpec(memory_space=pltpu.VMEM),
        ],
        interpret=interpret,
    )(w_flat, dp2, x0r, x1r)


def _tc_final(hp2, ir2, b, n_pad, interpret=False):
    rows = n_pad // 128

    def body(b_ref, hp_ref, ir_ref, o_ref):
        hsum = hp_ref[0:rows, :]
        for p in range(1, NC * NS):
            hsum = hsum + hp_ref[p * rows:(p + 1) * rows, :]
        o_ref[...] = hsum * ir_ref[...] + b_ref[0, 0]

    return pl.pallas_call(
        body,
        out_shape=jax.ShapeDtypeStruct((rows, 128), jnp.float32),
        in_specs=[
            pl.BlockSpec(memory_space=pltpu.SMEM),
            pl.BlockSpec(memory_space=pltpu.VMEM),
            pl.BlockSpec(memory_space=pltpu.VMEM),
        ],
        interpret=interpret,
    )(b.reshape(1, 1), hp2, ir2)


def kernel(x, edge_index, edge_weight, W, b):
    n = x.shape[0]
    e = edge_index.shape[1]
    n_pad = ((n + 1023) // 1024) * 1024
    rows = n_pad // 128
    pad = n_pad - n
    x0r = jnp.pad(x[:, 0], (0, pad)).reshape(rows, 128)
    x1r = jnp.pad(x[:, 1], (0, pad)).reshape(rows, 128)
    w_flat = W.reshape(1, 2)
    eflat = edge_index.reshape(2 * e)

    dp = _sc_degree_hist(eflat, e, n_pad)
    feat2, ir2 = _tc_feat(dp.reshape(NC * NS * rows, 128), x0r, x1r, w_flat,
                          n_pad)
    hp = _sc_aggregate(feat2.reshape(n_pad), eflat, edge_weight, e, n_pad)
    out2 = _tc_final(hp.reshape(NC * NS * rows, 128), ir2, b, n_pad)
    return out2.reshape(1, n_pad)[:, :n]
